# baseline (speedup 1.0000x reference)
import numpy as np
import jax
import jax.numpy as jnp
from jax import lax
from jax.experimental import pallas as pl
from jax.experimental.pallas import tpu as pltpu
from jax.experimental.pallas import tpu_sc as plsc

B = 4
NPTS = 70000
H = 28
W = 50
NVIEW = 24
SX = 1.0 / 16
SY = 1.0 / 16
WIN = 10000
NWIN = NPTS // WIN
GRID = 1408
EPS = 1e-6
UNROLL = 1

_GDN = lax.GatherDimensionNumbers(
    offset_dims=(), collapsed_slice_dims=(0,), start_index_map=(0,))


def _gather16(vec, idx):
    return lax.gather(vec, idx[:, None], _GDN, (1,),
                      mode=lax.GatherScatterMode.PROMISE_IN_BOUNDS)


def _bf16_round(x):
    i = plsc.bitcast(x, jnp.int32)
    r = (i + 0x7FFF + ((i >> 16) & 1)) & jnp.int32(-65536)
    return plsc.bitcast(r, jnp.float32)


def _sc_body(pts_hbm, mats_hbm, out_hbm, pbuf, mrow, dgrid, igrid):
    nc = 2
    wid = lax.axis_index("s") * nc + lax.axis_index("c")

    @pl.when(wid < NVIEW)
    def _():
        v = wid
        b = v // 6
        pltpu.sync_copy(mats_hbm.at[v], mrow)
        mv0 = mrow[pl.ds(0, 16)]
        mv1 = mrow[pl.ds(16, 16)]
        lane = lax.iota(jnp.int32, 16)

        def bc(vec, j):
            return _gather16(vec, jnp.full((16,), j, jnp.int32))

        a = [bc(mv0, j) for j in range(16)]
        g = [bc(mv1, j) for j in range(12)]
        cmax = bc(mv1, 12)
        va = a[0:12]
        pr = a[12:16] + g[0:8]
        cr = g[8:12]

        zf = jnp.zeros((16,), jnp.float32)

        def zinit(i, carry):
            dgrid[pl.ds(i * 16, 16)] = zf
            igrid[pl.ds(i * 16, 16)] = zf
            return carry

        lax.fori_loop(0, GRID // 16, zinit, 0)

        def point_group(o):
            px = pbuf[0, pl.ds(o, 16)]
            py = pbuf[1, pl.ds(o, 16)]
            pz = pbuf[2, pl.ds(o, 16)]
            il = pbuf[3, pl.ds(o, 16)]
            q0 = _bf16_round((va[0] * px + va[1] * py) + (va[2] * pz + va[3]))
            q1 = _bf16_round((va[4] * px + va[5] * py) + (va[6] * pz + va[7]))
            q2 = _bf16_round((va[8] * px + va[9] * py) + (va[10] * pz + va[11]))
            u = (pr[0] * q0 + pr[1] * q1) + (pr[2] * q2 + pr[3])
            vv = (pr[4] * q0 + pr[5] * q1) + (pr[6] * q2 + pr[7])
            w = (pr[8] * q0 + pr[9] * q1) + (pr[10] * q2 + pr[11])
            z = (cr[0] * q0 + cr[1] * q1) + (cr[2] * q2 + cr[3])
            wn = jnp.maximum(w, EPS)
            x = u / wn
            y = vv / wn
            valid = ((x > -0.5) & (x < W - 0.5)
                     & (y > -0.5) & (y < H - 0.5) & (z > 0.0))
            wc = jnp.minimum(jnp.maximum(w, 0.0), cmax)
            depth = jnp.where(valid, wc, 0.0)
            ilv = jnp.where(valid,
                            jnp.minimum(jnp.maximum(il, 0.0), 255.0), 0.0)
            ym = jnp.minimum(jnp.maximum(y, 0.0), H - 1.0).astype(jnp.int32)
            xm = jnp.minimum(jnp.maximum(x, 0.0), W - 1.0).astype(jnp.int32)
            c = ym * W + xm
            plsc.store_scatter(dgrid, [c], depth)
            plsc.store_scatter(igrid, [c], ilv)

        def point_iter(i, carry):
            for k in range(UNROLL):
                point_group(i * (16 * UNROLL) + k * 16)
            return carry

        for wnd in range(NWIN):
            base = wnd * WIN
            for f in range(4):
                pltpu.sync_copy(pts_hbm.at[b * 4 + f, pl.ds(base, WIN)],
                                pbuf.at[f])
            lax.fori_loop(0, WIN // (16 * UNROLL), point_iter, 0)

        pltpu.sync_copy(dgrid.at[pl.ds(0, H * W)], out_hbm.at[v, 0])
        pltpu.sync_copy(igrid.at[pl.ds(0, H * W)], out_hbm.at[v, 1])


@jax.jit
def _sc_call(pts_t, mats):
    mesh = plsc.VectorSubcoreMesh(core_axis_name="c", subcore_axis_name="s")
    return pl.kernel(
        _sc_body,
        mesh=mesh,
        compiler_params=pltpu.CompilerParams(use_tc_tiling_on_sc=False,
                                             needs_layout_passes=False),
        out_type=jax.ShapeDtypeStruct((NVIEW, 2, H * W), jnp.float32),
        scratch_types=[
            pltpu.VMEM((4, WIN), jnp.float32),
            pltpu.VMEM((32,), jnp.float32),
            pltpu.VMEM((GRID,), jnp.float32),
            pltpu.VMEM((GRID,), jnp.float32),
        ],
    )(pts_t, mats)


def _epi_body(raw_ref, hb_ref, out_ref):
    raw = raw_ref[...]
    hb = hb_ref[0, 0]
    rows = lax.broadcasted_iota(jnp.int32, (2 * NVIEW, H * W), 0)
    is_depth = (rows % 2) == 0
    out_ref[...] = jnp.where(is_depth, raw / hb,
                             jnp.log(raw + 1.0) / np.log(256.0))


@jax.jit
def _epi_call(raw, hb):
    return pl.pallas_call(
        _epi_body,
        out_shape=jax.ShapeDtypeStruct((2 * NVIEW, H * W), jnp.float32),
        in_specs=[
            pl.BlockSpec(memory_space=pltpu.VMEM),
            pl.BlockSpec(memory_space=pltpu.SMEM),
        ],
        out_specs=pl.BlockSpec(memory_space=pltpu.VMEM),
    )(raw, hb)


def kernel(pcloud_list, extrinsics, intrinsics, view, bev_size):
    f32 = jnp.float32
    pts = pcloud_list[1][:, 0, 0]
    pts_t = jnp.transpose(pts, (0, 2, 1)).reshape(4 * B, NPTS).astype(f32)
    rowf = jnp.arange(4 * B, dtype=jnp.int32) % 4
    ib = pts_t.view(jnp.int32)
    ib = (ib + 0x7FFF + ((ib >> 16) & 1)) & jnp.int32(-65536)
    pts_t = jnp.where((rowf == 3)[:, None], pts_t, ib.view(f32))

    view2 = view[:, 0, 0, :, :]
    vinv = jnp.linalg.inv(view2)
    intr = intrinsics.reshape(NVIEW, 3, 3)
    extrinsics_packed = extrinsics.reshape(NVIEW, 4, 4)
    K4 = jnp.tile(jnp.eye(4, dtype=intr.dtype)[None], (NVIEW, 1, 1))
    Ksc = intr.at[:, 0, :].multiply(SX).at[:, 1, :].multiply(SY)
    intr_scaled = K4.at[:, :3, :3].set(Ksc)
    r_transpose = jnp.swapaxes(extrinsics_packed[:, :3, :3], 1, 2)
    t_inv = -jnp.matmul(r_transpose, extrinsics_packed[:, :3, 3:4])
    extrinsic_inv = (extrinsics_packed.at[:, :3, :3].set(r_transpose)
                     .at[:, :3, 3:4].set(t_inv))
    pixel_from_car = jnp.matmul(intr_scaled, extrinsic_inv)

    bfr = lambda x: x.astype(jnp.bfloat16).astype(f32)
    vinv_b = bfr(jnp.repeat(vinv, 6, axis=0)[:, :3, :])
    pix_b = bfr(pixel_from_car[:, :3, :])
    cam_b = bfr(extrinsic_inv[:, 2, :])

    bev_side = bev_size[0].astype(f32)
    cmax = bev_side / 2 - 1
    tail = jnp.zeros((NVIEW, 4), f32).at[:, 0].set(cmax)
    mats = jnp.concatenate([vinv_b.reshape(NVIEW, 12),
                            pix_b.reshape(NVIEW, 12),
                            cam_b, tail], axis=1)

    raw = _sc_call(pts_t, mats)
    hb = (bev_side / 2).reshape(1, 1)
    out = _epi_call(raw.reshape(2 * NVIEW, H * W), hb)
    return out.reshape(NVIEW, 2, H, W)


# speedup vs baseline: 1.2176x; 1.2176x over previous
import numpy as np
import jax
import jax.numpy as jnp
from jax import lax
from jax.experimental import pallas as pl
from jax.experimental.pallas import tpu as pltpu
from jax.experimental.pallas import tpu_sc as plsc

B = 4
NPTS = 70000
H = 28
W = 50
NVIEW = 24
SX = 1.0 / 16
SY = 1.0 / 16
WIN = 2800
WPV = NPTS // WIN
GRID = 1408
EPS = 1e-6
SENT_I = 0x7FC00001

_GDN = lax.GatherDimensionNumbers(
    offset_dims=(), collapsed_slice_dims=(0,), start_index_map=(0,))


def _gather16(vec, idx):
    return lax.gather(vec, idx[:, None], _GDN, (1,),
                      mode=lax.GatherScatterMode.PROMISE_IN_BOUNDS)


def _bf16_round(x):
    i = plsc.bitcast(x, jnp.int32)
    r = (i + 0x7FFF + ((i >> 16) & 1)) & jnp.int32(-65536)
    return plsc.bitcast(r, jnp.float32)


def _schedule():
    merge = [[] for _ in range(NVIEW)]
    for s in range(32):
        w0 = 19 * s - max(s - 24, 0)
        cnt = 19 if s < 24 else 18
        views = []
        for wi in range(w0, w0 + cnt):
            v = wi // WPV
            if v not in views:
                views.append(v)
        for part, v in enumerate(views):
            merge[v].append((max(w0, WPV * v), s, part))
    for v in range(NVIEW):
        merge[v].sort()
    return [[(s, p) for (_, s, p) in merge[v]] for v in range(NVIEW)]


_MERGE = _schedule()


def _sc_body(pts_hbm, mats_hbm, out_hbm, pbuf, mrow, dgrid, igrid, dsem):
    nc = 2
    wid = lax.axis_index("s") * nc + lax.axis_index("c")
    w0 = 19 * wid - jnp.maximum(wid - 24, 0)
    cnt = jnp.where(wid < 24, 19, 18)
    lane = lax.iota(jnp.int32, 16)
    sent = plsc.bitcast(jnp.full((16,), SENT_I, jnp.int32), jnp.float32)

    def init_grids():
        def zi(i, carry):
            dgrid[pl.ds(i * 16, 16)] = sent
            igrid[pl.ds(i * 16, 16)] = sent
            return carry
        lax.fori_loop(0, GRID // 16, zi, 0)

    def flush(part):
        pltpu.sync_copy(dgrid, out_hbm.at[wid, part, 0])
        pltpu.sync_copy(igrid, out_hbm.at[wid, part, 1])

    def win_body(idx, carry):
        prev_v, part = carry
        wi = w0 + idx
        v = wi // WPV
        b = v // 6
        base = (wi % WPV) * WIN
        switch = v != prev_v
        flush_now = switch & (prev_v >= 0)

        @pl.when(flush_now)
        def _():
            flush(part)

        part = jnp.where(flush_now, part + 1, part)

        handles = [
            pltpu.async_copy(pts_hbm.at[b * 4 + f, pl.ds(base, WIN)],
                             pbuf.at[f], dsem)
            for f in range(4)
        ]

        @pl.when(switch)
        def _():
            init_grids()
            pltpu.sync_copy(mats_hbm.at[v], mrow)

        mv0 = mrow[pl.ds(0, 16)]
        mv1 = mrow[pl.ds(16, 16)]

        def bc(vec, j):
            return _gather16(vec, jnp.full((16,), j, jnp.int32))

        a = [bc(mv0, j) for j in range(16)]
        g = [bc(mv1, j) for j in range(13)]
        va = a[0:12]
        pr = a[12:16] + g[0:8]
        cr = g[8:12]
        cmax = g[12]

        for h in handles:
            h.wait()

        def point_iter(i, carry2):
            o = i * 16
            px = pbuf[0, pl.ds(o, 16)]
            py = pbuf[1, pl.ds(o, 16)]
            pz = pbuf[2, pl.ds(o, 16)]
            il = pbuf[3, pl.ds(o, 16)]
            q0 = _bf16_round((va[0] * px + va[1] * py) + (va[2] * pz + va[3]))
            q1 = _bf16_round((va[4] * px + va[5] * py) + (va[6] * pz + va[7]))
            q2 = _bf16_round((va[8] * px + va[9] * py) + (va[10] * pz + va[11]))
            u = (pr[0] * q0 + pr[1] * q1) + (pr[2] * q2 + pr[3])
            vv = (pr[4] * q0 + pr[5] * q1) + (pr[6] * q2 + pr[7])
            w = (pr[8] * q0 + pr[9] * q1) + (pr[10] * q2 + pr[11])
            z = (cr[0] * q0 + cr[1] * q1) + (cr[2] * q2 + cr[3])
            wn = jnp.maximum(w, EPS)
            x = u / wn
            y = vv / wn
            valid = ((x > -0.5) & (x < W - 0.5)
                     & (y > -0.5) & (y < H - 0.5) & (z > 0.0))
            wc = jnp.minimum(jnp.maximum(w, 0.0), cmax)
            depth = jnp.where(valid, wc, 0.0)
            ilv = jnp.where(valid,
                            jnp.minimum(jnp.maximum(il, 0.0), 255.0), 0.0)
            ym = jnp.minimum(jnp.maximum(y, 0.0), H - 1.0).astype(jnp.int32)
            xm = jnp.minimum(jnp.maximum(x, 0.0), W - 1.0).astype(jnp.int32)
            c = ym * W + xm
            plsc.store_scatter(dgrid, [c], depth)
            plsc.store_scatter(igrid, [c], ilv)
            return carry2

        lax.fori_loop(0, WIN // 16, point_iter, 0)
        return (v, part)

    prev_v, part = lax.fori_loop(0, cnt, win_body,
                                 (jnp.int32(-1), jnp.int32(0)))
    flush(part)


@jax.jit
def _sc_call(pts_t, mats):
    mesh = plsc.VectorSubcoreMesh(core_axis_name="c", subcore_axis_name="s")
    return pl.kernel(
        _sc_body,
        mesh=mesh,
        compiler_params=pltpu.CompilerParams(use_tc_tiling_on_sc=False,
                                             needs_layout_passes=False),
        out_type=jax.ShapeDtypeStruct((32, 2, 2, GRID), jnp.float32),
        scratch_types=[
            pltpu.VMEM((4, WIN), jnp.float32),
            pltpu.VMEM((32,), jnp.float32),
            pltpu.VMEM((GRID,), jnp.float32),
            pltpu.VMEM((GRID,), jnp.float32),
            pltpu.SemaphoreType.DMA,
        ],
    )(pts_t, mats)


def _epi_body(raw_ref, hb_ref, out_ref):
    hb = hb_ref[0, 0]
    for v in range(NVIEW):
        accd = acci = None
        for (s, p) in _MERGE[v]:
            rd = (s * 2 + p) * 2
            d = raw_ref[pl.ds(rd, 1), :]
            i_ = raw_ref[pl.ds(rd + 1, 1), :]
            wr = lax.bitcast_convert_type(d, jnp.int32) != SENT_I
            if accd is None:
                accd = jnp.where(wr, d, 0.0)
                acci = jnp.where(wr, i_, 0.0)
            else:
                accd = jnp.where(wr, d, accd)
                acci = jnp.where(wr, i_, acci)
        out_ref[pl.ds(2 * v, 1), :] = accd[:, :H * W] / hb
        out_ref[pl.ds(2 * v + 1, 1), :] = (
            jnp.log(acci[:, :H * W] + 1.0) / np.log(256.0))


@jax.jit
def _epi_call(raw, hb):
    return pl.pallas_call(
        _epi_body,
        out_shape=jax.ShapeDtypeStruct((2 * NVIEW, H * W), jnp.float32),
        in_specs=[
            pl.BlockSpec(memory_space=pltpu.VMEM),
            pl.BlockSpec(memory_space=pltpu.SMEM),
        ],
        out_specs=pl.BlockSpec(memory_space=pltpu.VMEM),
    )(raw, hb)


def kernel(pcloud_list, extrinsics, intrinsics, view, bev_size):
    f32 = jnp.float32
    pts = pcloud_list[1][:, 0, 0]
    pts_t = jnp.transpose(pts, (0, 2, 1)).reshape(4 * B, NPTS).astype(f32)
    rowf = jnp.arange(4 * B, dtype=jnp.int32) % 4
    ib = pts_t.view(jnp.int32)
    ib = (ib + 0x7FFF + ((ib >> 16) & 1)) & jnp.int32(-65536)
    pts_t = jnp.where((rowf == 3)[:, None], pts_t, ib.view(f32))

    view2 = view[:, 0, 0, :, :]
    vinv = jnp.linalg.inv(view2)
    intr = intrinsics.reshape(NVIEW, 3, 3)
    extrinsics_packed = extrinsics.reshape(NVIEW, 4, 4)
    K4 = jnp.tile(jnp.eye(4, dtype=intr.dtype)[None], (NVIEW, 1, 1))
    Ksc = intr.at[:, 0, :].multiply(SX).at[:, 1, :].multiply(SY)
    intr_scaled = K4.at[:, :3, :3].set(Ksc)
    r_transpose = jnp.swapaxes(extrinsics_packed[:, :3, :3], 1, 2)
    t_inv = -jnp.matmul(r_transpose, extrinsics_packed[:, :3, 3:4])
    extrinsic_inv = (extrinsics_packed.at[:, :3, :3].set(r_transpose)
                     .at[:, :3, 3:4].set(t_inv))
    pixel_from_car = jnp.matmul(intr_scaled, extrinsic_inv)

    bfr = lambda x: x.astype(jnp.bfloat16).astype(f32)
    vinv_b = bfr(jnp.repeat(vinv, 6, axis=0)[:, :3, :])
    pix_b = bfr(pixel_from_car[:, :3, :])
    cam_b = bfr(extrinsic_inv[:, 2, :])

    bev_side = bev_size[0].astype(f32)
    cmax = bev_side / 2 - 1
    tail = jnp.zeros((NVIEW, 4), f32).at[:, 0].set(cmax)
    mats = jnp.concatenate([vinv_b.reshape(NVIEW, 12),
                            pix_b.reshape(NVIEW, 12),
                            cam_b, tail], axis=1)

    raw = _sc_call(pts_t, mats)
    hb = (bev_side / 2).reshape(1, 1)
    out = _epi_call(raw.reshape(4 * 32, GRID), hb)
    return out.reshape(NVIEW, 2, H, W)


# speedup vs baseline: 1.3499x; 1.1087x over previous
import numpy as np
import jax
import jax.numpy as jnp
from jax import lax
from jax.experimental import pallas as pl
from jax.experimental.pallas import tpu as pltpu
from jax.experimental.pallas import tpu_sc as plsc

B = 4
NPTS = 70000
H = 28
W = 50
NVIEW = 24
SX = 1.0 / 16
SY = 1.0 / 16
WIN = 2800
WPV = NPTS // WIN
GRID = 1408
EPS = 1e-6
SENT_I = 0x7FC00001

_GDN = lax.GatherDimensionNumbers(
    offset_dims=(), collapsed_slice_dims=(0,), start_index_map=(0,))


def _gather16(vec, idx):
    return lax.gather(vec, idx[:, None], _GDN, (1,),
                      mode=lax.GatherScatterMode.PROMISE_IN_BOUNDS)


def _bf16_round(x):
    i = plsc.bitcast(x, jnp.int32)
    r = (i + 0x7FFF + ((i >> 16) & 1)) & jnp.int32(-65536)
    return plsc.bitcast(r, jnp.float32)


def _schedule():
    merge = [[] for _ in range(NVIEW)]
    for s in range(32):
        w0 = 19 * s - max(s - 24, 0)
        cnt = 19 if s < 24 else 18
        views = []
        for wi in range(w0, w0 + cnt):
            v = wi // WPV
            if v not in views:
                views.append(v)
        for part, v in enumerate(views):
            merge[v].append((max(w0, WPV * v), s, part))
    for v in range(NVIEW):
        merge[v].sort()
    return [[(s, p) for (_, s, p) in merge[v]] for v in range(NVIEW)]


_MERGE = _schedule()


def _sc_body(pts_hbm, mats_hbm, out_hbm, pbuf0, pbuf1, mrow, dgrid, igrid,
             dsem0, dsem1):
    nc = 2
    wid = lax.axis_index("s") * nc + lax.axis_index("c")
    w0 = 19 * wid - jnp.maximum(wid - 24, 0)
    cnt = jnp.where(wid < 24, 19, 18)
    lane = lax.iota(jnp.int32, 16)
    sent = plsc.bitcast(jnp.full((16,), SENT_I, jnp.int32), jnp.float32)
    bufs = (pbuf0, pbuf1)
    sems = (dsem0, dsem1)

    def issue(slot, b, base):
        for f in range(4):
            pltpu.async_copy(pts_hbm.at[b * 4 + f, pl.ds(base, WIN)],
                             bufs[slot].at[f], sems[slot])

    def drain(slot, b, base):
        for f in range(4):
            pltpu.make_async_copy(pts_hbm.at[b * 4 + f, pl.ds(base, WIN)],
                                  bufs[slot].at[f], sems[slot]).wait()

    def init_grids():
        def zi(i, carry):
            dgrid[pl.ds(i * 16, 16)] = sent
            igrid[pl.ds(i * 16, 16)] = sent
            return carry
        lax.fori_loop(0, GRID // 16, zi, 0)

    def flush(part):
        pltpu.sync_copy(dgrid, out_hbm.at[wid, part, 0])
        pltpu.sync_copy(igrid, out_hbm.at[wid, part, 1])

    def win_body(idx, carry):
        prev_v, part = carry
        wi = w0 + idx
        v = wi // WPV
        b = v // 6
        base = (wi % WPV) * WIN
        switch = v != prev_v
        flush_now = switch & (prev_v >= 0)

        @pl.when(flush_now)
        def _():
            flush(part)

        part = jnp.where(flush_now, part + 1, part)

        nidx = idx + 1
        has_next = nidx < cnt
        nwi = w0 + nidx
        nv2 = nwi // WPV
        nb = jnp.where(has_next, nv2 // 6, 0)
        nbase = jnp.where(has_next, (nwi % WPV) * WIN, 0)

        @pl.when(switch)
        def _():
            init_grids()
            pltpu.sync_copy(mats_hbm.at[v], mrow)

        mv0 = mrow[pl.ds(0, 16)]
        mv1 = mrow[pl.ds(16, 16)]

        def bc(vec, j):
            return _gather16(vec, jnp.full((16,), j, jnp.int32))

        a = [bc(mv0, j) for j in range(16)]
        g = [bc(mv1, j) for j in range(13)]
        va = a[0:12]
        pr = a[12:16] + g[0:8]
        cr = g[8:12]
        cmax = g[12]

        def point_iter_for(slot):
          buf = bufs[slot]

          def point_iter(i, carry2):
            o = i * 16
            px = buf[0, pl.ds(o, 16)]
            py = buf[1, pl.ds(o, 16)]
            pz = buf[2, pl.ds(o, 16)]
            il = buf[3, pl.ds(o, 16)]
            q0 = _bf16_round((va[0] * px + va[1] * py) + (va[2] * pz + va[3]))
            q1 = _bf16_round((va[4] * px + va[5] * py) + (va[6] * pz + va[7]))
            q2 = _bf16_round((va[8] * px + va[9] * py) + (va[10] * pz + va[11]))
            u = (pr[0] * q0 + pr[1] * q1) + (pr[2] * q2 + pr[3])
            vv = (pr[4] * q0 + pr[5] * q1) + (pr[6] * q2 + pr[7])
            w = (pr[8] * q0 + pr[9] * q1) + (pr[10] * q2 + pr[11])
            z = (cr[0] * q0 + cr[1] * q1) + (cr[2] * q2 + cr[3])
            wn = jnp.maximum(w, EPS)
            x = u / wn
            y = vv / wn
            valid = ((x > -0.5) & (x < W - 0.5)
                     & (y > -0.5) & (y < H - 0.5) & (z > 0.0))
            wc = jnp.minimum(jnp.maximum(w, 0.0), cmax)
            depth = jnp.where(valid, wc, 0.0)
            ilv = jnp.where(valid,
                            jnp.minimum(jnp.maximum(il, 0.0), 255.0), 0.0)
            ym = jnp.minimum(jnp.maximum(y, 0.0), H - 1.0).astype(jnp.int32)
            xm = jnp.minimum(jnp.maximum(x, 0.0), W - 1.0).astype(jnp.int32)
            c = ym * W + xm
            plsc.store_scatter(dgrid, [c], depth)
            plsc.store_scatter(igrid, [c], ilv)
            return carry2

          return point_iter

        def run_slot(slot):
            drain(slot, b, base)

            @pl.when(has_next)
            def _():
                issue(1 - slot, nb, nbase)

            lax.fori_loop(0, WIN // 16, point_iter_for(slot), 0)

        parity = lax.rem(idx, 2)

        @pl.when(parity == 0)
        def _():
            run_slot(0)

        @pl.when(parity == 1)
        def _():
            run_slot(1)

        return (v, part)

    v0 = w0 // WPV
    issue(0, v0 // 6, (w0 % WPV) * WIN)
    prev_v, part = lax.fori_loop(0, cnt, win_body,
                                 (jnp.int32(-1), jnp.int32(0)))
    flush(part)


@jax.jit
def _sc_call(pts_t, mats):
    mesh = plsc.VectorSubcoreMesh(core_axis_name="c", subcore_axis_name="s")
    return pl.kernel(
        _sc_body,
        mesh=mesh,
        compiler_params=pltpu.CompilerParams(use_tc_tiling_on_sc=False,
                                             needs_layout_passes=False),
        out_type=jax.ShapeDtypeStruct((32, 2, 2, GRID), jnp.float32),
        scratch_types=[
            pltpu.VMEM((4, WIN), jnp.float32),
            pltpu.VMEM((4, WIN), jnp.float32),
            pltpu.VMEM((32,), jnp.float32),
            pltpu.VMEM((GRID,), jnp.float32),
            pltpu.VMEM((GRID,), jnp.float32),
            pltpu.SemaphoreType.DMA,
            pltpu.SemaphoreType.DMA,
        ],
    )(pts_t, mats)


def _epi_body(raw_ref, hb_ref, out_ref):
    hb = hb_ref[0, 0]
    for v in range(NVIEW):
        accd = acci = None
        for (s, p) in _MERGE[v]:
            rd = (s * 2 + p) * 2
            d = raw_ref[pl.ds(rd, 1), :]
            i_ = raw_ref[pl.ds(rd + 1, 1), :]
            wr = lax.bitcast_convert_type(d, jnp.int32) != SENT_I
            if accd is None:
                accd = jnp.where(wr, d, 0.0)
                acci = jnp.where(wr, i_, 0.0)
            else:
                accd = jnp.where(wr, d, accd)
                acci = jnp.where(wr, i_, acci)
        out_ref[pl.ds(2 * v, 1), :] = accd[:, :H * W] / hb
        out_ref[pl.ds(2 * v + 1, 1), :] = (
            jnp.log(acci[:, :H * W] + 1.0) / np.log(256.0))


@jax.jit
def _epi_call(raw, hb):
    return pl.pallas_call(
        _epi_body,
        out_shape=jax.ShapeDtypeStruct((2 * NVIEW, H * W), jnp.float32),
        in_specs=[
            pl.BlockSpec(memory_space=pltpu.VMEM),
            pl.BlockSpec(memory_space=pltpu.SMEM),
        ],
        out_specs=pl.BlockSpec(memory_space=pltpu.VMEM),
    )(raw, hb)


def kernel(pcloud_list, extrinsics, intrinsics, view, bev_size):
    f32 = jnp.float32
    pts = pcloud_list[1][:, 0, 0]
    pts_t = jnp.transpose(pts, (0, 2, 1)).reshape(4 * B, NPTS).astype(f32)
    rowf = jnp.arange(4 * B, dtype=jnp.int32) % 4
    ib = pts_t.view(jnp.int32)
    ib = (ib + 0x7FFF + ((ib >> 16) & 1)) & jnp.int32(-65536)
    pts_t = jnp.where((rowf == 3)[:, None], pts_t, ib.view(f32))

    view2 = view[:, 0, 0, :, :]
    vinv = jnp.linalg.inv(view2)
    intr = intrinsics.reshape(NVIEW, 3, 3)
    extrinsics_packed = extrinsics.reshape(NVIEW, 4, 4)
    K4 = jnp.tile(jnp.eye(4, dtype=intr.dtype)[None], (NVIEW, 1, 1))
    Ksc = intr.at[:, 0, :].multiply(SX).at[:, 1, :].multiply(SY)
    intr_scaled = K4.at[:, :3, :3].set(Ksc)
    r_transpose = jnp.swapaxes(extrinsics_packed[:, :3, :3], 1, 2)
    t_inv = -jnp.matmul(r_transpose, extrinsics_packed[:, :3, 3:4])
    extrinsic_inv = (extrinsics_packed.at[:, :3, :3].set(r_transpose)
                     .at[:, :3, 3:4].set(t_inv))
    pixel_from_car = jnp.matmul(intr_scaled, extrinsic_inv)

    bfr = lambda x: x.astype(jnp.bfloat16).astype(f32)
    vinv_b = bfr(jnp.repeat(vinv, 6, axis=0)[:, :3, :])
    pix_b = bfr(pixel_from_car[:, :3, :])
    cam_b = bfr(extrinsic_inv[:, 2, :])

    bev_side = bev_size[0].astype(f32)
    cmax = bev_side / 2 - 1
    tail = jnp.zeros((NVIEW, 4), f32).at[:, 0].set(cmax)
    mats = jnp.concatenate([vinv_b.reshape(NVIEW, 12),
                            pix_b.reshape(NVIEW, 12),
                            cam_b, tail], axis=1)

    raw = _sc_call(pts_t, mats)
    hb = (bev_side / 2).reshape(1, 1)
    out = _epi_call(raw.reshape(4 * 32, GRID), hb)
    return out.reshape(NVIEW, 2, H, W)
